# Initial kernel scaffold; baseline (speedup 1.0000x reference)
import jax
import jax.numpy as jnp
from jax.experimental import pallas as pl


def kernel(x_tok, edge_index, edge_attr_tok, pert, gene, node_emb, edge_emb, W, b, att_src, att_dst, att_edge, We, ln_g, ln_b, mlp_W1, mlp_b1, mlp_W2, mlp_b2):
    raise NotImplementedError("write your pallas kernel here")


# speedup vs baseline: 14.3499x; 14.3499x over previous
import dataclasses
import functools

import jax
import jax.numpy as jnp
from jax import lax
from jax.experimental import pallas as pl
from jax.experimental.pallas import tpu as pltpu
from jax.experimental.pallas import tpu_sc as plsc

N = 10000
E = 320000
D = 128
HD = D // 2
FFN = 512
B = 4096
NC = 2
NS = 16
NW = NC * NS
TPE = E // NS
CHUNK = 80
NCHUNK = TPE // CHUNK
ZN = 624

_mesh = plsc.VectorSubcoreMesh(core_axis_name="c", subcore_axis_name="s")
_f32 = jnp.float32

_sc_params = pltpu.CompilerParams()
if "needs_layout_passes" in pltpu.CompilerParams.__dataclass_fields__:
    _sc_params = dataclasses.replace(_sc_params, needs_layout_passes=False)


def _sc_gather_rows(table, idx):
    n = idx.shape[0]
    d = table.shape[1]
    pad = (-n) % (NW * 8)
    p = n + pad
    if pad:
        idx = jnp.concatenate([idx, jnp.zeros((pad,), jnp.int32)])
    rpt = p // NW
    c = next(c for c in (128, 80, 64, 40, 32, 16, 8) if rpt % c == 0)

    @functools.partial(
        pl.kernel, mesh=_mesh,
        out_type=jax.ShapeDtypeStruct((p, d), _f32),
        scratch_types=[
            pltpu.VMEM((rpt,), jnp.int32),
            pltpu.VMEM((rpt, d), _f32),
            pltpu.SemaphoreType.DMA,
        ],
    )
    def k(table_h, idx_h, out_h, idx_v, rows_v, sem):
        wid = lax.axis_index("s") * NC + lax.axis_index("c")
        base = wid * rpt
        pltpu.sync_copy(idx_h.at[pl.ds(base, rpt)], idx_v)
        for j in range(rpt // c):
            pltpu.async_copy(
                table_h.at[idx_v.at[pl.ds(j * c, c)]],
                rows_v.at[pl.ds(j * c, c)], sem).wait()
        pltpu.sync_copy(rows_v, out_h.at[pl.ds(base, rpt)])

    out = k(table, idx)
    return out[:n] if pad else out


NH = N // NC
NGRP = TPE // 16
GPC = CHUNK // 16
ZR = 312


@functools.partial(
    pl.kernel, mesh=_mesh, compiler_params=_sc_params,
    out_type=[
        jax.ShapeDtypeStruct((N, D), _f32),
        jax.ShapeDtypeStruct((N,), _f32),
    ],
    scratch_types=[
        pltpu.VMEM((TPE,), jnp.int32),
        pltpu.VMEM((TPE,), _f32),
        pltpu.VMEM((N,), _f32),
        pltpu.VMEM((N,), _f32),
        pltpu.VMEM((16,), _f32),
        pltpu.VMEM((CHUNK, D), _f32),
        pltpu.VMEM((1, CHUNK), jnp.int32),
        pltpu.VMEM((1, CHUNK), jnp.int32),
        pltpu.VMEM((1, CHUNK), _f32),
        pltpu.VMEM((16,), _f32),
        pltpu.VMEM((NS, 16), _f32),
        pltpu.VMEM((1000,), _f32),
        pltpu.VMEM_SHARED((NH, D), _f32),
        pltpu.VMEM_SHARED((NH,), _f32),
        pltpu.VMEM_SHARED((NS, 16), _f32),
        pltpu.SemaphoreType.DMA,
    ],
)
def _sc_edge(pk_h, as_h, ad_h, tbl_h, hh_h,
             pout_h, s_h,
             pk_v, w_v, as_v, ad_v, tbl_v, rows_v, sc_v, dc_v, wc_v,
             mx_v, mxall_v, zs_v, acc_sh, s_sh, mx_sh, sem):
    cid = lax.axis_index("c")
    sid = lax.axis_index("s")
    lo = cid * NH

    pltpu.sync_copy(pk_h.at[pl.ds(sid * TPE, TPE)], pk_v)
    pltpu.sync_copy(as_h, as_v)
    pltpu.sync_copy(ad_h, ad_v)
    pltpu.sync_copy(tbl_h, tbl_v)

    @pl.loop(0, CHUNK)
    def _(r):
        for q in range(D // 16):
            rows_v[r, pl.ds(q * 16, 16)] = jnp.zeros((16,), _f32)

    @pl.loop(0, 1000, step=16)
    def _(i):
        zs_v[pl.ds(i, 16)] = jnp.zeros((16,), _f32)

    rbase = sid * ZR
    for rep in range(ZR // CHUNK):
        pltpu.sync_copy(rows_v, acc_sh.at[pl.ds(rbase + rep * CHUNK, CHUNK)])
    rem = ZR % CHUNK
    if rem:
        pltpu.sync_copy(rows_v.at[pl.ds(0, rem)],
                        acc_sh.at[pl.ds(rbase + ZR - rem, rem)])

    @pl.when(sid == NS - 1)
    def _():
        pltpu.sync_copy(rows_v.at[pl.ds(0, NH - NS * ZR)],
                        acc_sh.at[pl.ds(NS * ZR, NH - NS * ZR)])

    @pl.when(sid < NH // 1000)
    def _():
        pltpu.sync_copy(zs_v, s_sh.at[pl.ds(sid * 1000, 1000)])

    mask14 = jnp.full((16,), 0x3FFF, jnp.int32)

    mx_v[...] = jnp.full((16,), -3.0e38, _f32)

    @pl.loop(0, NGRP)
    def _(gi):
        p = pk_v[pl.ds(gi * 16, 16)]
        sv = lax.bitwise_and(p, mask14)
        dv = lax.bitwise_and(lax.shift_right_logical(p, 14), mask14)
        tv = lax.shift_right_logical(p, 28)
        a = (plsc.load_gather(as_v, [sv])
             + plsc.load_gather(ad_v, [dv])
             + plsc.load_gather(tbl_v, [tv]))
        a = jnp.maximum(a, 0.2 * a)
        w_v[pl.ds(gi * 16, 16)] = a
        mx_v[...] = jnp.maximum(mx_v[...], a)

    pltpu.sync_copy(mx_v, mx_sh.at[sid])
    plsc.subcore_barrier()
    pltpu.sync_copy(mx_sh, mxall_v)
    cur = mxall_v[0, :]
    for i in range(1, NS):
        cur = jnp.maximum(cur, mxall_v[i, :])
    g = jnp.max(cur)

    @pl.loop(0, NCHUNK)
    def _(j):
        for k in range(GPC):
            sl = pl.ds(k * 16, 16)
            p = pk_v[pl.ds(j * CHUNK + k * 16, 16)]
            sc_v[0, sl] = lax.bitwise_and(p, mask14)
            dv = lax.bitwise_and(lax.shift_right_logical(p, 14), mask14)
            dl = dv - lo
            ml = jnp.logical_and(dl >= 0, dl < NH)
            w = jnp.exp(w_v[pl.ds(j * CHUNK + k * 16, 16)] - g)
            dc_v[0, sl] = jnp.where(ml, dl, lax.shift_right_logical(dv, 1))
            wc_v[0, sl] = jnp.where(ml, w, 0.0)

        pltpu.async_copy(hh_h.at[sc_v.at[0]], rows_v, sem).wait()

        @pl.loop(0, CHUNK)
        def _(e):
            wb = plsc.load_gather(wc_v, [jnp.zeros((16,), jnp.int32),
                                         jnp.full((16,), e, jnp.int32)])
            for q in range(D // 16):
                sq = pl.ds(q * 16, 16)
                rows_v[e, sq] = rows_v[e, sq] * wb

        pltpu.sync_copy(rows_v, acc_sh.at[dc_v.at[0]], add=True)
        pltpu.sync_copy(wc_v.at[0], s_sh.at[dc_v.at[0]], add=True)

    plsc.subcore_barrier()

    pltpu.sync_copy(acc_sh.at[pl.ds(rbase, ZR)],
                    pout_h.at[pl.ds(lo + rbase, ZR)])

    @pl.when(sid == NS - 1)
    def _():
        pltpu.sync_copy(acc_sh.at[pl.ds(NS * ZR, NH - NS * ZR)],
                        pout_h.at[pl.ds(lo + NS * ZR, NH - NS * ZR)])

    @pl.when(sid < NH // 1000)
    def _():
        pltpu.sync_copy(s_sh.at[pl.ds(sid * 1000, 1000)], zs_v)
        pltpu.sync_copy(zs_v, s_h.at[pl.ds(lo + sid * 1000, 1000)])


_HI = lax.Precision.HIGHEST
_RB = 2000


def _prep(hn, W_l, a_s, a_d, e_emb, We_l, a_e, hh_ref, as_ref, ad_ref,
          tbl_ref):
    hh = jnp.dot(hn, W_l, precision=_HI)
    hh_ref[...] = hh
    as_ref[...] = jnp.sum(hh * a_s, -1, keepdims=True)
    ad_ref[...] = jnp.sum(hh * a_d, -1, keepdims=True)
    ef = jnp.dot(e_emb, We_l, precision=_HI)
    tbl_ref[...] = jnp.sum(ef * a_e, -1, keepdims=True)


def _ln(h, g, b):
    mu = jnp.mean(h, -1, keepdims=True)
    var = jnp.mean((h - mu) ** 2, -1, keepdims=True)
    return (h - mu) / jnp.sqrt(var + 1e-5) * g + b


def _tc_pre_body(h_ref, lg_ref, lb_ref, W_ref, as_ref, ad_ref, ee_ref, We_ref,
                 ae_ref, hh_ref, aso_ref, ado_ref, tbl_ref):
    hn = _ln(h_ref[...], lg_ref[...], lb_ref[...])
    _prep(hn, W_ref[...], as_ref[...], ad_ref[...], ee_ref[...], We_ref[...],
          ae_ref[...], hh_ref, aso_ref, ado_ref, tbl_ref)


def _tc_mid_body(last, p_ref, s_ref, resid_ref, b_ref, lg_ref, lb_ref,
                 W_ref, as_ref, ad_ref, ee_ref, We_ref, ae_ref,
                 h_ref, hh_ref=None, aso_ref=None, ado_ref=None,
                 tbl_ref=None):
    den = jnp.maximum(s_ref[...], 1e-30)
    out = p_ref[...] / den + b_ref[...]
    if last:
        h_ref[...] = resid_ref[...] + out
    else:
        h = resid_ref[...] + jax.nn.gelu(out)
        h_ref[...] = h
        hn = _ln(h, lg_ref[...], lb_ref[...])
        _prep(hn, W_ref[...], as_ref[...], ad_ref[...], ee_ref[...],
              We_ref[...], ae_ref[...], hh_ref, aso_ref, ado_ref, tbl_ref)


def _row_spec():
    return pl.BlockSpec((_RB, D), lambda i: (i, 0))


def _col_spec():
    return pl.BlockSpec((_RB, 1), lambda i: (i, 0))


def _const_spec(shape):
    nd = len(shape)
    return pl.BlockSpec(shape, lambda i: (0,) * nd)


_AUX_SHAPES = [
    jax.ShapeDtypeStruct((N, D), _f32),
    jax.ShapeDtypeStruct((N, 1), _f32),
    jax.ShapeDtypeStruct((N, 1), _f32),
    jax.ShapeDtypeStruct((16, 1), _f32),
]


def _aux_specs():
    return [_row_spec(), _col_spec(), _col_spec(), _const_spec((16, 1))]


def _tc_pre(h, ln_g, ln_b, W_l, a_s, a_d, e_emb, We_l, a_e):
    grid = (N // _RB,)
    in_specs = [_row_spec()] + [_const_spec((1, D))] * 2 + [
        _const_spec((D, D)), _const_spec((1, D)), _const_spec((1, D)),
        _const_spec((16, D)), _const_spec((D, D)), _const_spec((1, D))]
    return pl.pallas_call(
        _tc_pre_body, grid=grid, in_specs=in_specs, out_specs=_aux_specs(),
        out_shape=_AUX_SHAPES)(
            h, ln_g.reshape(1, D), ln_b.reshape(1, D), W_l,
            a_s.reshape(1, D), a_d.reshape(1, D), e_emb, We_l,
            a_e.reshape(1, D))


def _tc_mid(last, pout, s1, resid, b_l, ln_g, ln_b, W_l, a_s, a_d,
            e_emb, We_l, a_e):
    grid = (N // _RB,)
    out_shape = [jax.ShapeDtypeStruct((N, D), _f32)]
    out_specs = [_row_spec()]
    if not last:
        out_shape += _AUX_SHAPES
        out_specs += _aux_specs()
    in_specs = [
        _row_spec(), _col_spec(),
        _row_spec(), _const_spec((1, D)), _const_spec((1, D)),
        _const_spec((1, D)), _const_spec((D, D)), _const_spec((1, D)),
        _const_spec((1, D)), _const_spec((16, D)), _const_spec((D, D)),
        _const_spec((1, D))]
    return pl.pallas_call(
        functools.partial(_tc_mid_body, last), grid=grid, in_specs=in_specs,
        out_specs=out_specs, out_shape=out_shape)(
            pout, s1.reshape(N, 1), resid, b_l.reshape(1, D),
            ln_g.reshape(1, D), ln_b.reshape(1, D), W_l,
            a_s.reshape(1, D), a_d.reshape(1, D), e_emb, We_l,
            a_e.reshape(1, D))


def _tc_mlp_body(hp_ref, hg_ref, W1a_ref, W1b_ref, b1_ref, W2_ref, b2_ref,
                 o_ref):
    z = (jnp.dot(hp_ref[...], W1a_ref[...], precision=_HI)
         + jnp.dot(hg_ref[...], W1b_ref[...], precision=_HI)
         + b1_ref[...])
    hid = jax.nn.gelu(z)
    o_ref[...] = jnp.dot(hid, W2_ref[...], precision=_HI) + b2_ref[...]


def _tc_mlp(hp, hg, W1, b1, W2, b2):
    rb = 2048
    grid = (B // rb,)
    in_specs = [pl.BlockSpec((rb, D), lambda i: (i, 0)),
                pl.BlockSpec((rb, D), lambda i: (i, 0)),
                _const_spec((D, FFN)), _const_spec((D, FFN)),
                _const_spec((1, FFN)), _const_spec((FFN, 3)),
                _const_spec((1, 3))]
    out_specs = pl.BlockSpec((rb, 3), lambda i: (i, 0))
    return pl.pallas_call(
        _tc_mlp_body, grid=grid, in_specs=in_specs, out_specs=out_specs,
        out_shape=jax.ShapeDtypeStruct((B, 3), _f32))(
            hp, hg, W1[:D], W1[D:], b1.reshape(1, FFN), W2,
            b2.reshape(1, 3))


def kernel(x_tok, edge_index, edge_attr_tok, pert, gene, node_emb, edge_emb,
           W, b, att_src, att_dst, att_edge, We, ln_g, ln_b,
           mlp_W1, mlp_b1, mlp_W2, mlp_b2):
    x_tok = x_tok.astype(jnp.int32)
    pert = pert.astype(jnp.int32)
    gene = gene.astype(jnp.int32)
    src_e = edge_index[0].astype(jnp.int32)
    dst_e = edge_index[1].astype(jnp.int32)
    tok_e = edge_attr_tok.astype(jnp.int32)
    packed = src_e | (dst_e << 14) | (tok_e << 28)

    h = _sc_gather_rows(node_emb, x_tok)
    resid = h
    hh, as_c, ad_c, tbl_c = _tc_pre(h, ln_g[0], ln_b[0], W[0], att_src[0],
                                    att_dst[0], edge_emb, We[0], att_edge[0])
    for i in range(3):
        last = i == 2
        tbl = jnp.zeros((16,), _f32) if last else tbl_c.reshape(16)
        pout, s1 = _sc_edge(packed, as_c.reshape(N),
                            ad_c.reshape(N), tbl, hh)
        outs = _tc_mid(last, pout, s1, resid, b[i],
                       ln_g[min(i + 1, 2)], ln_b[min(i + 1, 2)],
                       W[min(i + 1, 2)], att_src[min(i + 1, 2)],
                       att_dst[min(i + 1, 2)], edge_emb, We[min(i + 1, 2)],
                       att_edge[min(i + 1, 2)])
        if last:
            h = outs[0]
        else:
            h, hh, as_c, ad_c, tbl_c = outs
            resid = h

    hp = _sc_gather_rows(h, pert)
    hg = _sc_gather_rows(h, gene)
    return _tc_mlp(hp, hg, mlp_W1, mlp_b1, mlp_W2, mlp_b2)


# speedup vs baseline: 23.6409x; 1.6475x over previous
import dataclasses
import functools

import jax
import jax.numpy as jnp
from jax import lax
from jax.experimental import pallas as pl
from jax.experimental.pallas import tpu as pltpu
from jax.experimental.pallas import tpu_sc as plsc

N = 10000
E = 320000
D = 128
HD = D // 2
FFN = 512
B = 4096
NC = 2
NS = 16
NW = NC * NS
TPE = E // NS
CHUNK = 80
NCHUNK = TPE // CHUNK
ZN = 624

_mesh = plsc.VectorSubcoreMesh(core_axis_name="c", subcore_axis_name="s")
_f32 = jnp.float32

_sc_params = pltpu.CompilerParams()
if "needs_layout_passes" in pltpu.CompilerParams.__dataclass_fields__:
    _sc_params = dataclasses.replace(_sc_params, needs_layout_passes=False)


def _sc_gather_rows(table, idx):
    n = idx.shape[0]
    d = table.shape[1]
    pad = (-n) % (NW * 8)
    p = n + pad
    if pad:
        idx = jnp.concatenate([idx, jnp.zeros((pad,), jnp.int32)])
    rpt = p // NW
    c = next(c for c in (128, 80, 64, 40, 32, 16, 8) if rpt % c == 0)

    @functools.partial(
        pl.kernel, mesh=_mesh,
        out_type=jax.ShapeDtypeStruct((p, d), _f32),
        scratch_types=[
            pltpu.VMEM((rpt,), jnp.int32),
            pltpu.VMEM((rpt, d), _f32),
            pltpu.SemaphoreType.DMA,
        ],
    )
    def k(table_h, idx_h, out_h, idx_v, rows_v, sem):
        wid = lax.axis_index("s") * NC + lax.axis_index("c")
        base = wid * rpt
        pltpu.sync_copy(idx_h.at[pl.ds(base, rpt)], idx_v)
        for j in range(rpt // c):
            pltpu.async_copy(
                table_h.at[idx_v.at[pl.ds(j * c, c)]],
                rows_v.at[pl.ds(j * c, c)], sem).wait()
        pltpu.sync_copy(rows_v, out_h.at[pl.ds(base, rpt)])

    out = k(table, idx)
    return out[:n] if pad else out


NH = N // NC
NGRP = TPE // 16
GPC = CHUNK // 16
ZR = 312


@functools.partial(
    pl.kernel, mesh=_mesh, compiler_params=_sc_params,
    out_type=[
        jax.ShapeDtypeStruct((N, D), _f32),
        jax.ShapeDtypeStruct((N,), _f32),
    ],
    scratch_types=[
        pltpu.VMEM((TPE,), jnp.int32),
        pltpu.VMEM((TPE,), _f32),
        pltpu.VMEM((N,), _f32),
        pltpu.VMEM((N,), _f32),
        pltpu.VMEM((16,), _f32),
        pltpu.VMEM((CHUNK, D), _f32),
        pltpu.VMEM((1, CHUNK), jnp.int32),
        pltpu.VMEM((1, CHUNK), jnp.int32),
        pltpu.VMEM((1, CHUNK), _f32),
        pltpu.VMEM((16,), _f32),
        pltpu.VMEM((NS, 16), _f32),
        pltpu.VMEM((1000,), _f32),
        pltpu.VMEM_SHARED((NH, D), _f32),
        pltpu.VMEM_SHARED((NH,), _f32),
        pltpu.VMEM_SHARED((NS, 16), _f32),
        pltpu.SemaphoreType.DMA,
    ],
)
def _sc_edge(pk_h, as_h, ad_h, tbl_h, hh_h,
             pout_h, s_h,
             pk_v, w_v, as_v, ad_v, tbl_v, rows_v, sc_v, dc_v, wc_v,
             mx_v, mxall_v, zs_v, acc_sh, s_sh, mx_sh, sem):
    cid = lax.axis_index("c")
    sid = lax.axis_index("s")
    lo = cid * NH

    pltpu.sync_copy(pk_h.at[pl.ds(sid * TPE, TPE)], pk_v)
    pltpu.sync_copy(as_h, as_v)
    pltpu.sync_copy(ad_h, ad_v)
    pltpu.sync_copy(tbl_h, tbl_v)

    @pl.loop(0, CHUNK)
    def _(r):
        for q in range(D // 16):
            rows_v[r, pl.ds(q * 16, 16)] = jnp.zeros((16,), _f32)

    @pl.loop(0, 1000, step=16)
    def _(i):
        zs_v[pl.ds(i, 16)] = jnp.zeros((16,), _f32)

    rbase = sid * ZR
    for rep in range(ZR // CHUNK):
        pltpu.sync_copy(rows_v, acc_sh.at[pl.ds(rbase + rep * CHUNK, CHUNK)])
    rem = ZR % CHUNK
    if rem:
        pltpu.sync_copy(rows_v.at[pl.ds(0, rem)],
                        acc_sh.at[pl.ds(rbase + ZR - rem, rem)])

    @pl.when(sid == NS - 1)
    def _():
        pltpu.sync_copy(rows_v.at[pl.ds(0, NH - NS * ZR)],
                        acc_sh.at[pl.ds(NS * ZR, NH - NS * ZR)])

    @pl.when(sid < NH // 1000)
    def _():
        pltpu.sync_copy(zs_v, s_sh.at[pl.ds(sid * 1000, 1000)])

    mask14 = jnp.full((16,), 0x3FFF, jnp.int32)

    mx_v[...] = jnp.full((16,), -3.0e38, _f32)

    def _p1(gi, off):
        p = pk_v[pl.ds(gi * 16, 16)]
        sv = lax.bitwise_and(p, mask14)
        dv = lax.bitwise_and(lax.shift_right_logical(p, 14), mask14)
        tv = lax.shift_right_logical(p, 28)
        a = (plsc.load_gather(as_v, [sv])
             + plsc.load_gather(ad_v, [dv])
             + plsc.load_gather(tbl_v, [tv]))
        a = jnp.maximum(a, 0.2 * a)
        mx_v[...] = jnp.maximum(mx_v[...], a)
        dl = dv - lo
        ml = jnp.logical_and(dl >= 0, dl < NH)
        plsc.store_compressed(pk_v.at[pl.ds(off, 16)],
                              lax.bitwise_or(sv, lax.shift_left(dl, 14)),
                              mask=ml)
        plsc.store_compressed(w_v.at[pl.ds(off, 16)], a, mask=ml)
        return off + jnp.max(plsc.all_reduce_population_count(ml))

    cnt = lax.fori_loop(0, NGRP, _p1, 0, unroll=False)

    for k in range(GPC):
        pk_v[pl.ds(cnt + k * 16, 16)] = jnp.zeros((16,), jnp.int32)
        w_v[pl.ds(cnt + k * 16, 16)] = jnp.full((16,), -3.0e38, _f32)
    nch = (cnt + CHUNK - 1) // CHUNK

    pltpu.sync_copy(mx_v, mx_sh.at[sid])
    plsc.subcore_barrier()
    pltpu.sync_copy(mx_sh, mxall_v)
    cur = mxall_v[0, :]
    for i in range(1, NS):
        cur = jnp.maximum(cur, mxall_v[i, :])
    g = jnp.max(cur)

    @pl.loop(0, nch)
    def _(j):
        for k in range(GPC):
            sl = pl.ds(k * 16, 16)
            p = pk_v[pl.ds(j * CHUNK + k * 16, 16)]
            sc_v[0, sl] = lax.bitwise_and(p, mask14)
            dc_v[0, sl] = lax.shift_right_logical(p, 14)
            wc_v[0, sl] = jnp.exp(w_v[pl.ds(j * CHUNK + k * 16, 16)] - g)

        pltpu.async_copy(hh_h.at[sc_v.at[0]], rows_v, sem).wait()

        @pl.loop(0, CHUNK)
        def _(e):
            wb = plsc.load_gather(wc_v, [jnp.zeros((16,), jnp.int32),
                                         jnp.full((16,), e, jnp.int32)])
            for q in range(D // 16):
                sq = pl.ds(q * 16, 16)
                rows_v[e, sq] = rows_v[e, sq] * wb

        pltpu.sync_copy(rows_v, acc_sh.at[dc_v.at[0]], add=True)
        pltpu.sync_copy(wc_v.at[0], s_sh.at[dc_v.at[0]], add=True)

    plsc.subcore_barrier()

    pltpu.sync_copy(acc_sh.at[pl.ds(rbase, ZR)],
                    pout_h.at[pl.ds(lo + rbase, ZR)])

    @pl.when(sid == NS - 1)
    def _():
        pltpu.sync_copy(acc_sh.at[pl.ds(NS * ZR, NH - NS * ZR)],
                        pout_h.at[pl.ds(lo + NS * ZR, NH - NS * ZR)])

    @pl.when(sid < NH // 1000)
    def _():
        pltpu.sync_copy(s_sh.at[pl.ds(sid * 1000, 1000)], zs_v)
        pltpu.sync_copy(zs_v, s_h.at[pl.ds(lo + sid * 1000, 1000)])


_HI = None
_RB = 2000


def _prep(hn, W_l, a_s, a_d, e_emb, We_l, a_e, hh_ref, as_ref, ad_ref,
          tbl_ref):
    hh = jnp.dot(hn, W_l, precision=_HI)
    hh_ref[...] = hh
    as_ref[...] = jnp.sum(hh * a_s, -1, keepdims=True)
    ad_ref[...] = jnp.sum(hh * a_d, -1, keepdims=True)
    ef = jnp.dot(e_emb, We_l, precision=_HI)
    tbl_ref[...] = jnp.sum(ef * a_e, -1, keepdims=True)


def _ln(h, g, b):
    mu = jnp.mean(h, -1, keepdims=True)
    var = jnp.mean((h - mu) ** 2, -1, keepdims=True)
    return (h - mu) / jnp.sqrt(var + 1e-5) * g + b


def _tc_pre_body(h_ref, lg_ref, lb_ref, W_ref, as_ref, ad_ref, ee_ref, We_ref,
                 ae_ref, hh_ref, aso_ref, ado_ref, tbl_ref):
    hn = _ln(h_ref[...], lg_ref[...], lb_ref[...])
    _prep(hn, W_ref[...], as_ref[...], ad_ref[...], ee_ref[...], We_ref[...],
          ae_ref[...], hh_ref, aso_ref, ado_ref, tbl_ref)


def _tc_mid_body(last, p_ref, s_ref, resid_ref, b_ref, lg_ref, lb_ref,
                 W_ref, as_ref, ad_ref, ee_ref, We_ref, ae_ref,
                 h_ref, hh_ref=None, aso_ref=None, ado_ref=None,
                 tbl_ref=None):
    den = jnp.maximum(s_ref[...], 1e-30)
    out = p_ref[...] / den + b_ref[...]
    if last:
        h_ref[...] = resid_ref[...] + out
    else:
        h = resid_ref[...] + jax.nn.gelu(out)
        h_ref[...] = h
        hn = _ln(h, lg_ref[...], lb_ref[...])
        _prep(hn, W_ref[...], as_ref[...], ad_ref[...], ee_ref[...],
              We_ref[...], ae_ref[...], hh_ref, aso_ref, ado_ref, tbl_ref)


def _row_spec():
    return pl.BlockSpec((_RB, D), lambda i: (i, 0))


def _col_spec():
    return pl.BlockSpec((_RB, 1), lambda i: (i, 0))


def _const_spec(shape):
    nd = len(shape)
    return pl.BlockSpec(shape, lambda i: (0,) * nd)


_AUX_SHAPES = [
    jax.ShapeDtypeStruct((N, D), _f32),
    jax.ShapeDtypeStruct((N, 1), _f32),
    jax.ShapeDtypeStruct((N, 1), _f32),
    jax.ShapeDtypeStruct((16, 1), _f32),
]


def _aux_specs():
    return [_row_spec(), _col_spec(), _col_spec(), _const_spec((16, 1))]


def _tc_pre(h, ln_g, ln_b, W_l, a_s, a_d, e_emb, We_l, a_e):
    grid = (N // _RB,)
    in_specs = [_row_spec()] + [_const_spec((1, D))] * 2 + [
        _const_spec((D, D)), _const_spec((1, D)), _const_spec((1, D)),
        _const_spec((16, D)), _const_spec((D, D)), _const_spec((1, D))]
    return pl.pallas_call(
        _tc_pre_body, grid=grid, in_specs=in_specs, out_specs=_aux_specs(),
        out_shape=_AUX_SHAPES)(
            h, ln_g.reshape(1, D), ln_b.reshape(1, D), W_l,
            a_s.reshape(1, D), a_d.reshape(1, D), e_emb, We_l,
            a_e.reshape(1, D))


def _tc_mid(last, pout, s1, resid, b_l, ln_g, ln_b, W_l, a_s, a_d,
            e_emb, We_l, a_e):
    grid = (N // _RB,)
    out_shape = [jax.ShapeDtypeStruct((N, D), _f32)]
    out_specs = [_row_spec()]
    if not last:
        out_shape += _AUX_SHAPES
        out_specs += _aux_specs()
    in_specs = [
        _row_spec(), _col_spec(),
        _row_spec(), _const_spec((1, D)), _const_spec((1, D)),
        _const_spec((1, D)), _const_spec((D, D)), _const_spec((1, D)),
        _const_spec((1, D)), _const_spec((16, D)), _const_spec((D, D)),
        _const_spec((1, D))]
    return pl.pallas_call(
        functools.partial(_tc_mid_body, last), grid=grid, in_specs=in_specs,
        out_specs=out_specs, out_shape=out_shape)(
            pout, s1.reshape(N, 1), resid, b_l.reshape(1, D),
            ln_g.reshape(1, D), ln_b.reshape(1, D), W_l,
            a_s.reshape(1, D), a_d.reshape(1, D), e_emb, We_l,
            a_e.reshape(1, D))


def _tc_mlp_body(hp_ref, hg_ref, W1a_ref, W1b_ref, b1_ref, W2_ref, b2_ref,
                 o_ref):
    z = (jnp.dot(hp_ref[...], W1a_ref[...], precision=_HI)
         + jnp.dot(hg_ref[...], W1b_ref[...], precision=_HI)
         + b1_ref[...])
    hid = jax.nn.gelu(z)
    o_ref[...] = jnp.dot(hid, W2_ref[...], precision=_HI) + b2_ref[...]


def _tc_mlp(hp, hg, W1, b1, W2, b2):
    rb = 2048
    grid = (B // rb,)
    in_specs = [pl.BlockSpec((rb, D), lambda i: (i, 0)),
                pl.BlockSpec((rb, D), lambda i: (i, 0)),
                _const_spec((D, FFN)), _const_spec((D, FFN)),
                _const_spec((1, FFN)), _const_spec((FFN, 3)),
                _const_spec((1, 3))]
    out_specs = pl.BlockSpec((rb, 3), lambda i: (i, 0))
    return pl.pallas_call(
        _tc_mlp_body, grid=grid, in_specs=in_specs, out_specs=out_specs,
        out_shape=jax.ShapeDtypeStruct((B, 3), _f32))(
            hp, hg, W1[:D], W1[D:], b1.reshape(1, FFN), W2,
            b2.reshape(1, 3))


def kernel(x_tok, edge_index, edge_attr_tok, pert, gene, node_emb, edge_emb,
           W, b, att_src, att_dst, att_edge, We, ln_g, ln_b,
           mlp_W1, mlp_b1, mlp_W2, mlp_b2):
    x_tok = x_tok.astype(jnp.int32)
    pert = pert.astype(jnp.int32)
    gene = gene.astype(jnp.int32)
    src_e = edge_index[0].astype(jnp.int32)
    dst_e = edge_index[1].astype(jnp.int32)
    tok_e = edge_attr_tok.astype(jnp.int32)
    packed = src_e | (dst_e << 14) | (tok_e << 28)

    h = _sc_gather_rows(node_emb, x_tok)
    resid = h
    hh, as_c, ad_c, tbl_c = _tc_pre(h, ln_g[0], ln_b[0], W[0], att_src[0],
                                    att_dst[0], edge_emb, We[0], att_edge[0])
    for i in range(3):
        last = i == 2
        tbl = jnp.zeros((16,), _f32) if last else tbl_c.reshape(16)
        pout, s1 = _sc_edge(packed, as_c.reshape(N),
                            ad_c.reshape(N), tbl, hh)
        outs = _tc_mid(last, pout, s1, resid, b[i],
                       ln_g[min(i + 1, 2)], ln_b[min(i + 1, 2)],
                       W[min(i + 1, 2)], att_src[min(i + 1, 2)],
                       att_dst[min(i + 1, 2)], edge_emb, We[min(i + 1, 2)],
                       att_edge[min(i + 1, 2)])
        if last:
            h = outs[0]
        else:
            h, hh, as_c, ad_c, tbl_c = outs
            resid = h

    hp = _sc_gather_rows(h, pert)
    hg = _sc_gather_rows(h, gene)
    return _tc_mlp(hp, hg, mlp_W1, mlp_b1, mlp_W2, mlp_b2)
